# Initial kernel scaffold; baseline (speedup 1.0000x reference)
import jax
import jax.numpy as jnp
from jax.experimental import pallas as pl


def kernel(x, table):
    raise NotImplementedError("write your pallas kernel here")


# speedup vs baseline: 1.5585x; 1.5585x over previous
import functools

import jax
import jax.numpy as jnp
from jax import lax
from jax.experimental import pallas as pl
from jax.experimental.pallas import tpu as pltpu
from jax.experimental.pallas import tpu_sc as plsc

D = 32
L = 16
SUB = 128
NC = 2
NS = 16
NW = NC * NS
B = 4096 * 200
BPW = B // NW
NSTREAM = 25
C = NSTREAM * SUB
NCHUNK = BPW // C


def _body(x_hbm, table_hbm, out_hbm, idx_v, rows_v, flag_v, sem):
    wid = lax.axis_index("s") * NC + lax.axis_index("c")
    base = wid * BPW

    def chunk(k, carry):
        off = base + k * C
        pltpu.sync_copy(x_hbm.at[pl.ds(off, C)], idx_v)
        copies = []
        for j in range(NSTREAM):
            copies.append(
                pltpu.async_copy(
                    table_hbm.at[idx_v.at[pl.ds(j * SUB, SUB)]],
                    rows_v.at[pl.ds(j * SUB, SUB)],
                    sem,
                )
            )
        for cp in copies:
            cp.wait()

        def mstep(i, mv):
            return jnp.minimum(mv, idx_v[pl.ds(i * L, L)])

        mv = lax.fori_loop(
            0, C // L, mstep, jnp.full((L,), jnp.int32(2**30), jnp.int32)
        )
        flag_v[pl.ds(0, L)] = jnp.zeros((L,), jnp.int32)
        plsc.store_scatter(
            flag_v,
            [jnp.zeros((L,), jnp.int32)],
            jnp.ones((L,), jnp.int32),
            mask=mv == 0,
        )
        has_pad = flag_v[pl.ds(0, L)][0] > 0

        @pl.when(has_pad)
        def _fix():
            zeros = jnp.zeros((L,), jnp.float32)
            lane = lax.iota(jnp.int32, L)

            def fix(g, c2):
                idx16 = idx_v[pl.ds(g * L, L)]
                zmask = idx16 == 0
                rowids = g * L + lane
                for col in range(D):
                    plsc.store_scatter(
                        rows_v,
                        [rowids, jnp.full((L,), jnp.int32(col), jnp.int32)],
                        zeros,
                        mask=zmask,
                    )
                return c2

            lax.fori_loop(0, C // L, fix, 0)

        pltpu.sync_copy(rows_v, out_hbm.at[pl.ds(off, C)])
        return carry

    lax.fori_loop(0, NCHUNK, chunk, 0)


@jax.jit
def _embedding(x1d, table):
    mesh = plsc.VectorSubcoreMesh(core_axis_name="c", subcore_axis_name="s")
    f = pl.kernel(
        _body,
        out_type=jax.ShapeDtypeStruct((B, D), jnp.float32),
        mesh=mesh,
        scratch_types=[
            pltpu.VMEM((C,), jnp.int32),
            pltpu.VMEM((C, D), jnp.float32),
            pltpu.VMEM((L,), jnp.int32),
            pltpu.SemaphoreType.DMA,
        ],
        compiler_params=pltpu.CompilerParams(
            needs_layout_passes=False, use_tc_tiling_on_sc=False
        ),
    )
    return f(x1d, table)


def kernel(x, table):
    bsz, seq = x.shape
    out = _embedding(x.reshape(B), table)
    return out.reshape(bsz, seq, D)


# speedup vs baseline: 1.5797x; 1.0136x over previous
import functools

import jax
import jax.numpy as jnp
from jax import lax
from jax.experimental import pallas as pl
from jax.experimental.pallas import tpu as pltpu
from jax.experimental.pallas import tpu_sc as plsc

D = 32
L = 16
SUB = 128
NC = 2
NS = 16
NW = NC * NS
B = 4096 * 200
BPW = B // NW
NSTREAM = 10
C = NSTREAM * SUB
NCHUNK = BPW // C


def _body(x_hbm, table_hbm, out_hbm, idx_v, rows0_v, rows1_v, flag_v,
          semg0, semg1, semo):
    wid = lax.axis_index("s") * NC + lax.axis_index("c")
    base = wid * BPW

    pltpu.sync_copy(x_hbm.at[pl.ds(base, BPW)], idx_v)

    def fire_gather(c, buf, sem):
        for s in range(NSTREAM):
            pltpu.async_copy(
                table_hbm.at[idx_v.at[pl.ds(c * C + s * SUB, SUB)]],
                buf.at[pl.ds(s * SUB, SUB)],
                sem,
            )

    def drain_gather(buf, sem):
        pltpu.make_async_copy(table_hbm.at[pl.ds(0, C)], buf, sem).wait()

    def fire_out(c, buf):
        pltpu.async_copy(buf, out_hbm.at[pl.ds(base + c * C, C)], semo)

    def drain_out(buf):
        pltpu.make_async_copy(buf, out_hbm.at[pl.ds(0, C)], semo).wait()

    def fix_chunk(c, buf):
        def mstep(i, mv):
            return jnp.minimum(mv, idx_v[pl.ds(c * C + i * L, L)])

        mv = lax.fori_loop(
            0, C // L, mstep, jnp.full((L,), jnp.int32(2**30), jnp.int32)
        )
        flag_v[pl.ds(0, L)] = jnp.zeros((L,), jnp.int32)
        plsc.store_scatter(
            flag_v,
            [jnp.zeros((L,), jnp.int32)],
            jnp.ones((L,), jnp.int32),
            mask=mv == 0,
        )
        has_pad = flag_v[pl.ds(0, L)][0] > 0

        @pl.when(has_pad)
        def _fix():
            zeros = jnp.zeros((L,), jnp.float32)
            lane = lax.iota(jnp.int32, L)

            def fix(g, c2):
                idx16 = idx_v[pl.ds(c * C + g * L, L)]
                zmask = idx16 == 0
                rowids = g * L + lane
                for col in range(D):
                    plsc.store_scatter(
                        buf,
                        [rowids, jnp.full((L,), jnp.int32(col), jnp.int32)],
                        zeros,
                        mask=zmask,
                    )
                return c2

            lax.fori_loop(0, C // L, fix, 0)

    fire_gather(0, rows0_v, semg0)

    def pair(jj, carry):
        c0 = 2 * jj
        c1 = c0 + 1

        @pl.when(jj > 0)
        def _():
            drain_out(rows1_v)
        fire_gather(c1, rows1_v, semg1)
        drain_gather(rows0_v, semg0)
        fix_chunk(c0, rows0_v)
        fire_out(c0, rows0_v)

        @pl.when(jj < NCHUNK // 2 - 1)
        def _():
            drain_out(rows0_v)
            fire_gather(c1 + 1, rows0_v, semg0)
        drain_gather(rows1_v, semg1)
        fix_chunk(c1, rows1_v)
        fire_out(c1, rows1_v)
        return carry

    lax.fori_loop(0, NCHUNK // 2, pair, 0)
    drain_out(rows0_v)
    drain_out(rows1_v)


@jax.jit
def _embedding(x1d, table):
    mesh = plsc.VectorSubcoreMesh(core_axis_name="c", subcore_axis_name="s")
    f = pl.kernel(
        _body,
        out_type=jax.ShapeDtypeStruct((B, D), jnp.float32),
        mesh=mesh,
        scratch_types=[
            pltpu.VMEM((BPW,), jnp.int32),
            pltpu.VMEM((C, D), jnp.float32),
            pltpu.VMEM((C, D), jnp.float32),
            pltpu.VMEM((L,), jnp.int32),
            pltpu.SemaphoreType.DMA,
            pltpu.SemaphoreType.DMA,
            pltpu.SemaphoreType.DMA,
        ],
        compiler_params=pltpu.CompilerParams(
            needs_layout_passes=False, use_tc_tiling_on_sc=False
        ),
    )
    return f(x1d, table)


def kernel(x, table):
    bsz, seq = x.shape
    out = _embedding(x.reshape(B), table)
    return out.reshape(bsz, seq, D)
